# baseline (speedup 1.0000x reference)
import functools

import jax
import jax.numpy as jnp
from jax.experimental import pallas as pl
from jax.experimental.pallas import tpu as pltpu
from jax.experimental.pallas import tpu_sc as plsc

_GATHER_WINDOW = 128
_BB = 64


def _sc_gather_pairs(pairs):
    tables = [t for t, _ in pairs]
    idxs = [i for _, i in pairs]
    d = tables[0].shape[1]
    mesh = plsc.VectorSubcoreMesh(core_axis_name="core", subcore_axis_name="subcore")
    out_types = tuple(
        jax.ShapeDtypeStruct((idx.shape[1], d), t.dtype) for t, idx in pairs
    )
    np_ = len(pairs)

    @functools.partial(
        pl.kernel,
        out_type=out_types,
        mesh=mesh,
        compiler_params=pltpu.CompilerParams(use_tc_tiling_on_sc=False),
    )
    def k(*refs):
        table_refs = refs[:np_]
        idx_refs = refs[np_: 2 * np_]
        out_refs = refs[2 * np_:]

        for t_hbm, i_hbm, o_hbm in zip(table_refs, idx_refs, out_refs):
            def body(i_vmem, o_vmem, t_hbm=t_hbm):
                pltpu.sync_copy(t_hbm.at[i_vmem.at[0]], o_vmem)

            n = i_hbm.shape[1]
            pltpu.emit_pipeline(
                body,
                grid=(n // _GATHER_WINDOW,),
                in_specs=[pl.BlockSpec((1, _GATHER_WINDOW), index_map=lambda i: (0, i))],
                out_specs=[pl.BlockSpec((_GATHER_WINDOW, d), index_map=lambda i: (i, 0))],
                core_axis_name=("core", "subcore"),
                dimension_semantics=(pltpu.PARALLEL,),
            )(i_hbm, o_hbm)

    res = k(*tables, *idxs)
    return list(res) if isinstance(res, (tuple, list)) else [res]


def _side_embedding(gh, gt, g0, rels, rel_emb, w1, w2, w3, wagg, bmat):
    d = gh.shape[-1]
    s = gh.shape[2]
    bb = gh.shape[1]
    w1f = w1[...]
    waggf = wagg[...]
    bmatf = bmat[...]
    w1a = w1f[0:d, :]
    w1b = w1f[d : 2 * d, :]
    r1tab = jnp.dot(rel_emb[...], w1b, preferred_element_type=jnp.float32)
    w3r = jnp.broadcast_to(w3[...], (d, d))
    w2f = w2[...]
    b1 = bmatf[0:1, :]
    b2 = bmatf[1:2, :]
    bagg = bmatf[2:3, :]
    b3 = bmatf[3:4, 0:1]
    nr = rel_emb.shape[0]
    lane_iota = jax.lax.broadcasted_iota(jnp.int32, (1, nr), 1)

    def layer(h3, t3, rl):
        hf = h3.reshape(bb * s, d)
        oh = (rl == lane_iota).astype(jnp.float32)
        x = (jnp.dot(hf, w1a, preferred_element_type=jnp.float32)
             + jnp.dot(oh, r1tab, preferred_element_type=jnp.float32) + b1)
        x = jnp.maximum(x, 0.0)
        x = jnp.maximum(jnp.dot(x, w2f, preferred_element_type=jnp.float32) + b2, 0.0)
        lg = jax.nn.sigmoid(jnp.dot(x, w3r, preferred_element_type=jnp.float32) + b3)
        p = jnp.exp(lg.reshape(bb, s, d))
        p = p / jnp.sum(p, axis=1, keepdims=True)
        return jnp.sum(p * t3, axis=1)

    e0 = jnp.mean(g0[...], axis=1)
    e1 = layer(gh[0], gt[0], rels[0])
    e2 = layer(gh[1], gt[1], rels[1])
    y = (jnp.dot(e0, waggf[0:d, :], preferred_element_type=jnp.float32)
         + jnp.dot(e1, waggf[d : 2 * d, :], preferred_element_type=jnp.float32)
         + jnp.dot(e2, waggf[2 * d : 3 * d, :], preferred_element_type=jnp.float32)
         + bagg)
    return jax.nn.sigmoid(y)


def _tc_u_body(gh, gt, g0, rels, rel_emb, w1, w2, w3, wagg, bmat, out):
    out[...] = _side_embedding(gh, gt, g0, rels, rel_emb, w1, w2, w3, wagg, bmat)


def _tc_v_body(gh, gt, g0, rels, rel_emb, w1, w2, w3, wagg, bmat, eu, out):
    ev = _side_embedding(gh, gt, g0, rels, rel_emb, w1, w2, w3, wagg, bmat)
    out[...] = jax.nn.sigmoid(jnp.sum(eu[...] * ev, axis=1, keepdims=True))


def _tc_side_specs(b, s, d, nr, with_eu):
    bb = _BB
    grid = (b // bb,)
    in_specs = [
        pl.BlockSpec((2, bb, s, d), lambda i: (0, i, 0, 0)),
        pl.BlockSpec((2, bb, s, d), lambda i: (0, i, 0, 0)),
        pl.BlockSpec((bb, s, d), lambda i: (i, 0, 0)),
        pl.BlockSpec((2, bb * s, 1), lambda i: (0, i, 0)),
        pl.BlockSpec((nr, d), lambda i: (0, 0)),
        pl.BlockSpec((2 * d, d), lambda i: (0, 0)),
        pl.BlockSpec((d, d), lambda i: (0, 0)),
        pl.BlockSpec((d, 1), lambda i: (0, 0)),
        pl.BlockSpec((3 * d, d), lambda i: (0, 0)),
        pl.BlockSpec((8, d), lambda i: (0, 0)),
    ]
    if with_eu:
        in_specs.append(pl.BlockSpec((bb, d), lambda i: (i, 0)))
        out_specs = pl.BlockSpec((bb, 1), lambda i: (i, 0))
        out_shape = jax.ShapeDtypeStruct((b, 1), jnp.float32)
    else:
        out_specs = pl.BlockSpec((bb, d), lambda i: (i, 0))
        out_shape = jax.ShapeDtypeStruct((b, d), jnp.float32)
    return grid, in_specs, out_specs, out_shape


def kernel(u_entities, u_heads, u_relations, u_tails,
           v_entities, v_heads, v_relations, v_tails,
           entity_emb, rec_emb, rel_emb,
           W1, b1, W2, b2, W3, b3, Wagg, bagg):
    b, s = u_entities.shape
    d = entity_emb.shape[1]
    nr = rel_emb.shape[0]
    ls = u_heads.shape[0]

    g_uh, g_ut, g_ue = _sc_gather_pairs([
        (entity_emb, u_heads.reshape(1, -1)),
        (entity_emb, u_tails.reshape(1, -1)),
        (rec_emb, u_entities.reshape(1, -1)),
    ])
    g_vh, g_vt, g_ve = _sc_gather_pairs([
        (entity_emb, v_heads.reshape(1, -1)),
        (entity_emb, v_tails.reshape(1, -1)),
        (entity_emb, v_entities.reshape(1, -1)),
    ])

    bmat = (jnp.zeros((8, d), jnp.float32)
            .at[0].set(b1).at[1].set(b2).at[2].set(bagg).at[3].set(b3[0]))
    weights = (rel_emb, W1, W2, W3, Wagg, bmat)

    grid, in_specs_u, out_specs_u, out_shape_u = _tc_side_specs(b, s, d, nr, False)
    eu = pl.pallas_call(
        _tc_u_body, grid=grid, in_specs=in_specs_u,
        out_specs=out_specs_u, out_shape=out_shape_u,
    )(g_uh.reshape(ls, b, s, d), g_ut.reshape(ls, b, s, d),
      g_ue.reshape(b, s, d), u_relations.reshape(ls, b * s, 1), *weights)

    grid, in_specs_v, out_specs_v, out_shape_v = _tc_side_specs(b, s, d, nr, True)
    out = pl.pallas_call(
        _tc_v_body, grid=grid, in_specs=in_specs_v,
        out_specs=out_specs_v, out_shape=out_shape_v,
    )(g_vh.reshape(ls, b, s, d), g_vt.reshape(ls, b, s, d),
      g_ve.reshape(b, s, d), v_relations.reshape(ls, b * s, 1), *weights, eu)
    return out.reshape(-1)
